# baseline (speedup 1.0000x reference)
import functools

import jax
import jax.numpy as jnp
from jax import lax
from jax.experimental import pallas as pl
from jax.experimental.pallas import tpu as pltpu
from jax.experimental.pallas import tpu_sc as plsc

_F = 26
_DP = 16
_SB = 32


def _sc_gather(tables, idx4, drain):
    NW, NBLK = idx4.shape[0], idx4.shape[1]
    B = NW * NBLK * 8 * 128 // 32
    D = tables.shape[2]
    mesh = plsc.VectorSubcoreMesh(core_axis_name="c", subcore_axis_name="s")
    NC = mesh.num_cores
    NB = B // (NC * mesh.num_subcores)

    @functools.partial(
        pl.kernel,
        out_type=jax.ShapeDtypeStruct((B, _F * _DP), jnp.float32),
        mesh=mesh,
        scratch_types=[
            pltpu.VMEM((NBLK, 8, 128), jnp.int32),
            pltpu.VMEM((2, _F, 8, D), jnp.float32),
            pltpu.VMEM((_SB, _F * _DP), jnp.float32),
            pltpu.SemaphoreType.DMA,
            pltpu.SemaphoreType.DMA,
        ],
        compiler_params=pltpu.CompilerParams(needs_layout_passes=False),
    )
    def gather_kernel(
        table_hbm, idx_hbm, drain_hbm, out_hbm, idx_v, ring, stage, sem0, sem1
    ):
        wid = lax.axis_index("s") * NC + lax.axis_index("c")
        b0 = wid * NB
        pltpu.sync_copy(idx_hbm.at[wid], idx_v)
        lane16 = jax.lax.iota(jnp.int32, 16)
        m10 = lane16 < 10
        sems = (sem0, sem1)
        slot0 = lane16
        slot1 = jnp.minimum(lane16 + 16, _F - 1)
        col0 = lane16 * _DP
        col1 = (lane16 + 16) * _DP

        def fire(c, rb, sem):
            off = c * 32
            q, r, l0 = off // 1024, (off // 128) % 8, off % 128
            wa = idx_v[q, r, pl.ds(l0, 16)]
            wb = idx_v[q, r, pl.ds(l0 + 16, 16)]
            for f in range(_F):
                row = (wa if f < 16 else wb)[f % 16]
                rg = pl.multiple_of(row - (row & 7), 8)
                pltpu.async_copy(
                    table_hbm.at[f, pl.ds(rg, 8)], ring.at[rb, f], sem
                )

        def extract(c, rb):
            si_v = jnp.broadcast_to(c % _SB, (16,)).astype(jnp.int32)
            rb_v = jnp.full((16,), rb, dtype=jnp.int32)
            off = c * 32
            q, r, l0 = off // 1024, (off // 128) % 8, off % 128
            v16a = idx_v[q, r, pl.ds(l0, 16)]
            v16b = idx_v[q, r, pl.ds(l0 + 16, 16)]
            rma = jnp.bitwise_and(v16a, 7)
            rmb = jnp.bitwise_and(v16b, 7)
            for d in range(D):
                d_v = jnp.full((16,), d, dtype=jnp.int32)
                va = plsc.load_gather(ring, [rb_v, slot0, rma, d_v])
                plsc.store_scatter(stage, [si_v, col0 + d], va)
                vb = plsc.load_gather(ring, [rb_v, slot1, rmb, d_v])
                plsc.store_scatter(stage, [si_v, col1 + d], vb, mask=m10)

        def drain_wait(rb, sem):
            pltpu.make_async_copy(drain_hbm, ring.at[rb], sem).wait()

        def flush(c):
            @pl.when(c % _SB == _SB - 1)
            def _():
                start = pl.multiple_of(b0 + (c - (_SB - 1)), _SB)
                pltpu.sync_copy(stage, out_hbm.at[pl.ds(start, _SB)])

        fire(0, 0, sem0)
        fire(1, 1, sem1)

        def body(cc, carry):
            for k in range(2):
                c = cc * 2 + k
                drain_wait(k, sems[k])
                extract(c, k)

                @pl.when(c + 2 < NB)
                def _():
                    fire(c + 2, k, sems[k])

                flush(c)
            return carry

        lax.fori_loop(0, NB // 2, body, 0)

    return gather_kernel(tables, idx4, drain)


def _tc_matmul(emb16, W16, b2):
    B, K = emb16.shape
    H = W16.shape[1]
    BM = 1024

    def mm(emb_ref, w_ref, b_ref, out_ref):
        lane = lax.broadcasted_iota(jnp.int32, (BM, K), 1)
        e = jnp.where(lane % _DP < 10, emb_ref[...], 0.0)
        out_ref[...] = (
            jnp.dot(e, w_ref[...], preferred_element_type=jnp.float32) + b_ref[...]
        )

    return pl.pallas_call(
        mm,
        grid=(B // BM,),
        in_specs=[
            pl.BlockSpec((BM, K), lambda i: (i, 0)),
            pl.BlockSpec((K, H), lambda i: (0, 0)),
            pl.BlockSpec((1, H), lambda i: (0, 0)),
        ],
        out_specs=pl.BlockSpec((BM, H), lambda i: (i, 0)),
        out_shape=jax.ShapeDtypeStruct((B, H), jnp.float32),
    )(emb16, W16, b2)


def kernel(cov, tables, W, b):
    B, F = cov.shape
    _, V, D = tables.shape
    H = W.shape[1]
    idx2 = jnp.pad(cov.astype(jnp.int32), ((0, 0), (0, 32 - F)))
    idx4 = idx2.reshape(32, B * 32 // 32 // 1024, 8, 128)
    drain = jnp.zeros((F, 8, D), dtype=jnp.float32)

    emb16 = _sc_gather(tables, idx4, drain)

    W16 = jnp.pad(W.reshape(F, D, H), ((0, 0), (0, _DP - D), (0, 0)))
    W16 = W16.reshape(F * _DP, H)
    return _tc_matmul(emb16, W16, b.reshape(1, H))


# speedup vs baseline: 1.0590x; 1.0590x over previous
import functools

import jax
import jax.numpy as jnp
from jax import lax
from jax.experimental import pallas as pl
from jax.experimental.pallas import tpu as pltpu
from jax.experimental.pallas import tpu_sc as plsc

_F = 26
_DP = 16
_SB = 32


def _sc_gather(tables, idx4, drain):
    NW, NBLK = idx4.shape[0], idx4.shape[1]
    B = NW * NBLK * 8 * 128 // 32
    D = tables.shape[2]
    mesh = plsc.VectorSubcoreMesh(core_axis_name="c", subcore_axis_name="s")
    NC = mesh.num_cores
    NB = B // (NC * mesh.num_subcores)

    @functools.partial(
        pl.kernel,
        out_type=jax.ShapeDtypeStruct((B, _F * _DP), jnp.float32),
        mesh=mesh,
        scratch_types=[
            pltpu.VMEM((NBLK, 8, 128), jnp.int32),
            pltpu.VMEM((3, _F, 8, D), jnp.float32),
            pltpu.VMEM((_SB, _F * _DP), jnp.float32),
            pltpu.SemaphoreType.DMA,
            pltpu.SemaphoreType.DMA,
            pltpu.SemaphoreType.DMA,
        ],
        compiler_params=pltpu.CompilerParams(needs_layout_passes=False),
    )
    def gather_kernel(
        table_hbm, idx_hbm, drain_hbm, out_hbm, idx_v, ring, stage, sem0, sem1, sem2
    ):
        wid = lax.axis_index("s") * NC + lax.axis_index("c")
        b0 = wid * NB
        pltpu.sync_copy(idx_hbm.at[wid], idx_v)
        lane16 = jax.lax.iota(jnp.int32, 16)
        m10 = lane16 < 10
        sems = (sem0, sem1, sem2)
        slot0 = lane16
        slot1 = jnp.minimum(lane16 + 16, _F - 1)
        col0 = lane16 * _DP
        col1 = (lane16 + 16) * _DP

        def fire(c, rb, sem):
            off = c * 32
            q, r, l0 = off // 1024, (off // 128) % 8, off % 128
            wa = idx_v[q, r, pl.ds(l0, 16)]
            wb = idx_v[q, r, pl.ds(l0 + 16, 16)]
            for f in range(_F):
                row = (wa if f < 16 else wb)[f % 16]
                rg = pl.multiple_of(row - (row & 7), 8)
                pltpu.async_copy(
                    table_hbm.at[f, pl.ds(rg, 8)], ring.at[rb, f], sem
                )

        def extract(c, rb):
            si_v = jnp.broadcast_to(c % _SB, (16,)).astype(jnp.int32)
            rb_v = jnp.full((16,), rb, dtype=jnp.int32)
            off = c * 32
            q, r, l0 = off // 1024, (off // 128) % 8, off % 128
            v16a = idx_v[q, r, pl.ds(l0, 16)]
            v16b = idx_v[q, r, pl.ds(l0 + 16, 16)]
            rma = jnp.bitwise_and(v16a, 7)
            rmb = jnp.bitwise_and(v16b, 7)
            for d in range(D):
                d_v = jnp.full((16,), d, dtype=jnp.int32)
                va = plsc.load_gather(ring, [rb_v, slot0, rma, d_v])
                plsc.store_scatter(stage, [si_v, col0 + d], va)
                vb = plsc.load_gather(ring, [rb_v, slot1, rmb, d_v])
                plsc.store_scatter(stage, [si_v, col1 + d], vb, mask=m10)

        def drain_wait(rb, sem):
            pltpu.make_async_copy(drain_hbm, ring.at[rb], sem).wait()

        def _do_flush(c):
            start = pl.multiple_of(b0 + (c - (_SB - 1)), _SB)
            pltpu.sync_copy(stage, out_hbm.at[pl.ds(start, _SB)])

        def flush(c):
            if isinstance(c, int):
                if c % _SB == _SB - 1:
                    _do_flush(c)
                return

            @pl.when(c % _SB == _SB - 1)
            def _():
                _do_flush(c)

        fire(0, 0, sem0)
        fire(1, 1, sem1)
        fire(2, 2, sem2)

        def body(cc, carry):
            for k in range(3):
                c = cc * 3 + k
                drain_wait(k, sems[k])
                extract(c, k)

                @pl.when(c + 3 < NB)
                def _():
                    fire(c + 3, k, sems[k])

                flush(c)
            return carry

        lax.fori_loop(0, NB // 3, body, 0)
        for c in range(3 * (NB // 3), NB):
            k = c % 3
            drain_wait(k, sems[k])
            extract(c, k)
            flush(c)

    return gather_kernel(tables, idx4, drain)


def _tc_matmul(emb16, W16, b2):
    B, K = emb16.shape
    H = W16.shape[1]
    BM = 1024

    def mm(emb_ref, w_ref, b_ref, out_ref):
        lane = lax.broadcasted_iota(jnp.int32, (BM, K), 1)
        e = jnp.where(lane % _DP < 10, emb_ref[...], 0.0)
        out_ref[...] = (
            jnp.dot(e, w_ref[...], preferred_element_type=jnp.float32) + b_ref[...]
        )

    return pl.pallas_call(
        mm,
        grid=(B // BM,),
        in_specs=[
            pl.BlockSpec((BM, K), lambda i: (i, 0)),
            pl.BlockSpec((K, H), lambda i: (0, 0)),
            pl.BlockSpec((1, H), lambda i: (0, 0)),
        ],
        out_specs=pl.BlockSpec((BM, H), lambda i: (i, 0)),
        out_shape=jax.ShapeDtypeStruct((B, H), jnp.float32),
    )(emb16, W16, b2)


def kernel(cov, tables, W, b):
    B, F = cov.shape
    _, V, D = tables.shape
    H = W.shape[1]
    idx2 = jnp.pad(cov.astype(jnp.int32), ((0, 0), (0, 32 - F)))
    idx4 = idx2.reshape(32, B * 32 // 32 // 1024, 8, 128)
    drain = jnp.zeros((F, 8, D), dtype=jnp.float32)

    emb16 = _sc_gather(tables, idx4, drain)

    W16 = jnp.pad(W.reshape(F, D, H), ((0, 0), (0, _DP - D), (0, 0)))
    W16 = W16.reshape(F * _DP, H)
    return _tc_matmul(emb16, W16, b.reshape(1, H))


# speedup vs baseline: 1.2952x; 1.2231x over previous
import functools

import jax
import jax.numpy as jnp
from jax import lax
from jax.experimental import pallas as pl
from jax.experimental.pallas import tpu as pltpu
from jax.experimental.pallas import tpu_sc as plsc

_F = 26
_DP = 16
_SB = 32


def _sc_gather(tables, idx4, drain, V):
    NW, NBLK = idx4.shape[0], idx4.shape[1]
    B = NW * NBLK * 8 * 128 // 32
    D = tables.shape[1]
    mesh = plsc.VectorSubcoreMesh(core_axis_name="c", subcore_axis_name="s")
    NC = mesh.num_cores
    NB = B // (NC * mesh.num_subcores)

    @functools.partial(
        pl.kernel,
        out_type=jax.ShapeDtypeStruct((B, _F * _DP), jnp.float32),
        mesh=mesh,
        scratch_types=[
            pltpu.VMEM((NBLK, 8, 128), jnp.int32),
            pltpu.VMEM((3, _F, 8, D), jnp.float32),
            pltpu.VMEM((_SB, _F * _DP), jnp.float32),
            pltpu.SemaphoreType.DMA,
            pltpu.SemaphoreType.DMA,
            pltpu.SemaphoreType.DMA,
        ],
        compiler_params=pltpu.CompilerParams(needs_layout_passes=False),
    )
    def gather_kernel(
        table_hbm, idx_hbm, drain_hbm, out_hbm, idx_v, ring, stage, sem0, sem1, sem2
    ):
        wid = lax.axis_index("s") * NC + lax.axis_index("c")
        b0 = wid * NB
        pltpu.sync_copy(idx_hbm.at[wid], idx_v)
        lane16 = jax.lax.iota(jnp.int32, 16)
        m10 = lane16 < 10
        sems = (sem0, sem1, sem2)
        slot0 = lane16
        slot1 = jnp.minimum(lane16 + 16, _F - 1)
        col0 = lane16 * _DP
        col1 = (lane16 + 16) * _DP

        def fire(c, rb, sem):
            off = c * 32
            q, r, l0 = off // 1024, (off // 128) % 8, off % 128
            wa = idx_v[q, r, pl.ds(l0, 16)]
            wb = idx_v[q, r, pl.ds(l0 + 16, 16)]
            for f in range(_F):
                row = (wa if f < 16 else wb)[f % 16]
                rg = pl.multiple_of(f * V + (row - (row & 7)), 8)
                pltpu.async_copy(
                    table_hbm.at[pl.ds(rg, 8)], ring.at[rb, f], sem
                )

        def extract(c, rb):
            si_v = jnp.broadcast_to(c % _SB, (16,)).astype(jnp.int32)
            rb_v = jnp.full((16,), rb, dtype=jnp.int32)
            off = c * 32
            q, r, l0 = off // 1024, (off // 128) % 8, off % 128
            v16a = idx_v[q, r, pl.ds(l0, 16)]
            v16b = idx_v[q, r, pl.ds(l0 + 16, 16)]
            rma = jnp.bitwise_and(v16a, 7)
            rmb = jnp.bitwise_and(v16b, 7)
            for d in range(D):
                d_v = jnp.full((16,), d, dtype=jnp.int32)
                va = plsc.load_gather(ring, [rb_v, slot0, rma, d_v])
                plsc.store_scatter(stage, [si_v, col0 + d], va)
                vb = plsc.load_gather(ring, [rb_v, slot1, rmb, d_v])
                plsc.store_scatter(stage, [si_v, col1 + d], vb, mask=m10)

        def drain_wait(rb, sem):
            pltpu.make_async_copy(drain_hbm, ring.at[rb], sem).wait()

        def _do_flush(c):
            start = pl.multiple_of(b0 + (c - (_SB - 1)), _SB)
            pltpu.sync_copy(stage, out_hbm.at[pl.ds(start, _SB)])

        def flush(c):
            if isinstance(c, int):
                if c % _SB == _SB - 1:
                    _do_flush(c)
                return

            @pl.when(c % _SB == _SB - 1)
            def _():
                _do_flush(c)

        fire(0, 0, sem0)
        fire(1, 1, sem1)
        fire(2, 2, sem2)

        def body(cc, carry):
            for k in range(3):
                c = cc * 3 + k
                drain_wait(k, sems[k])
                extract(c, k)

                @pl.when(c + 3 < NB)
                def _():
                    fire(c + 3, k, sems[k])

                flush(c)
            return carry

        lax.fori_loop(0, NB // 3, body, 0)
        for c in range(3 * (NB // 3), NB):
            k = c % 3
            drain_wait(k, sems[k])
            extract(c, k)
            flush(c)

    return gather_kernel(tables, idx4, drain)


def _tc_matmul(emb16, W16, b2):
    B, K = emb16.shape
    H = W16.shape[1]
    BM = 1024

    def mm(emb_ref, w_ref, b_ref, out_ref):
        lane = lax.broadcasted_iota(jnp.int32, (BM, K), 1)
        e = jnp.where(lane % _DP < 10, emb_ref[...], 0.0)
        out_ref[...] = (
            jnp.dot(e, w_ref[...], preferred_element_type=jnp.float32) + b_ref[...]
        )

    return pl.pallas_call(
        mm,
        grid=(B // BM,),
        in_specs=[
            pl.BlockSpec((BM, K), lambda i: (i, 0)),
            pl.BlockSpec((K, H), lambda i: (0, 0)),
            pl.BlockSpec((1, H), lambda i: (0, 0)),
        ],
        out_specs=pl.BlockSpec((BM, H), lambda i: (i, 0)),
        out_shape=jax.ShapeDtypeStruct((B, H), jnp.float32),
    )(emb16, W16, b2)


def kernel(cov, tables, W, b):
    B, F = cov.shape
    _, V, D = tables.shape
    H = W.shape[1]
    idx2 = jnp.pad(cov.astype(jnp.int32), ((0, 0), (0, 32 - F)))
    idx4 = idx2.reshape(32, B * 32 // 32 // 1024, 8, 128)
    drain = jnp.zeros((F, 8, D), dtype=jnp.float32)

    emb16 = _sc_gather(tables.reshape(F * V, D), idx4, drain, V)

    W16 = jnp.pad(W.reshape(F, D, H), ((0, 0), (0, _DP - D), (0, 0)))
    W16 = W16.reshape(F * _DP, H)
    return _tc_matmul(emb16, W16, b.reshape(1, H))
